# Initial kernel scaffold; baseline (speedup 1.0000x reference)
import jax
import jax.numpy as jnp
from jax.experimental import pallas as pl


def kernel(x, edge_index, beta1, beta2):
    raise NotImplementedError("write your pallas kernel here")


# speedup vs baseline: 5.1676x; 5.1676x over previous
import functools

import jax
import jax.numpy as jnp
import numpy as np
from jax import lax
from jax.experimental import pallas as pl
from jax.experimental.pallas import tpu as pltpu
from jax.experimental.pallas import tpu_sc as plsc

N = 10000
E = 320000
D = 128
NPAD = 10240

NC = 2
NS = 16
NW = NC * NS
EPW = E // NW
C = 80
NCHUNK = EPW // C
ROWS_PER_TILE = NPAD // NS

_mesh = plsc.VectorSubcoreMesh(core_axis_name="c", subcore_axis_name="s")
_LANE = np.arange(16, dtype=np.int32)


def _norm_body(x_ref, o_ref):
    xb = x_ref[...]
    n = jnp.sqrt(jnp.sum(xb * xb, axis=1, keepdims=True))
    o_ref[...] = xb / (n + 1e-12)


def _normalize(x):
    return pl.pallas_call(
        _norm_body,
        out_shape=jax.ShapeDtypeStruct((N, D), jnp.float32),
        grid=(25,),
        in_specs=[pl.BlockSpec((400, D), lambda i: (i, 0))],
        out_specs=pl.BlockSpec((400, D), lambda i: (i, 0)),
    )(x)


def _comb_norm_body(a_ref, b_ref, h_ref, hn_ref):
    h = a_ref[...] + b_ref[...]
    h = jnp.where(h > 0, h, 0.3 * h)
    h_ref[...] = h
    n = jnp.sqrt(jnp.sum(h * h, axis=1, keepdims=True))
    hn_ref[...] = h / (n + 1e-12)


def _combine_leaky_normalize(a, b):
    return pl.pallas_call(
        _comb_norm_body,
        out_shape=(jax.ShapeDtypeStruct((NPAD, D), jnp.float32),
                   jax.ShapeDtypeStruct((NPAD, D), jnp.float32)),
        grid=(20,),
        in_specs=[pl.BlockSpec((512, D), lambda i: (i, 0)),
                  pl.BlockSpec((512, D), lambda i: (i, 0))],
        out_specs=(pl.BlockSpec((512, D), lambda i: (i, 0)),
                   pl.BlockSpec((512, D), lambda i: (i, 0))),
    )(a, b)


def _comb_body(a_ref, b_ref, o_ref):
    h = a_ref[...] + b_ref[...]
    o_ref[...] = jnp.where(h > 0, h, 0.3 * h)


def _combine_leaky(a, b):
    return pl.pallas_call(
        _comb_body,
        out_shape=jax.ShapeDtypeStruct((NPAD, D), jnp.float32),
        grid=(20,),
        in_specs=[pl.BlockSpec((512, D), lambda i: (i, 0)),
                  pl.BlockSpec((512, D), lambda i: (i, 0))],
        out_specs=pl.BlockSpec((512, D), lambda i: (i, 0)),
    )(a, b)


def _dinv_body(d_ref, o_ref):
    o_ref[...] = 1.0 / (d_ref[0] + d_ref[1] + 1e-12)


def _dinv(denom2):
    d3 = denom2.reshape(2, 80, 128)
    out = pl.pallas_call(
        _dinv_body,
        out_shape=jax.ShapeDtypeStruct((80, 128), jnp.float32),
    )(d3)
    return out.reshape(NPAD)


def _edge_pass1(xn, src, dst, bvec):
    @functools.partial(
        pl.kernel,
        out_type=(jax.ShapeDtypeStruct((E,), jnp.float32),
                  jax.ShapeDtypeStruct((2, NPAD), jnp.float32)),
        mesh=_mesh,
        compiler_params=pltpu.CompilerParams(needs_layout_passes=False),
        scratch_types=[
            pltpu.VMEM((C,), jnp.int32),
            pltpu.VMEM((C,), jnp.int32),
            pltpu.VMEM((C, D), jnp.float32),
            pltpu.VMEM((C, D), jnp.float32),
            pltpu.VMEM((C,), jnp.float32),
            pltpu.VMEM((16,), jnp.float32),
            pltpu.VMEM((NPAD // NS,), jnp.float32),
            pltpu.VMEM_SHARED((NPAD,), jnp.float32),
            pltpu.SemaphoreType.DMA,
            pltpu.SemaphoreType.DMA,
        ],
    )
    def k(xn_h, src_h, dst_h, bvec_h, e_out, denom_out,
          sidx, didx, srows, drows, ebuf, bv, zbuf, denom_sp,
          sem1, sem2):
        c = lax.axis_index("c")
        s = lax.axis_index("s")
        wid = s * NC + c

        zslice = NPAD // NS
        for i in range(zslice // 16):
            zbuf[pl.ds(i * 16, 16)] = jnp.zeros((16,), jnp.float32)
        pltpu.sync_copy(zbuf, denom_sp.at[pl.ds(s * zslice, zslice)])
        pltpu.sync_copy(bvec_h, bv)
        plsc.subcore_barrier()

        def chunk(kk, carry):
            base = wid * EPW + kk * C
            pltpu.sync_copy(src_h.at[pl.ds(base, C)], sidx)
            pltpu.sync_copy(dst_h.at[pl.ds(base, C)], didx)
            cp1 = pltpu.async_copy(xn_h.at[sidx], srows, sem1)
            cp2 = pltpu.async_copy(xn_h.at[didx], drows, sem2)
            cp1.wait()
            cp2.wait()
            bb = bv[...]
            mb = -jnp.abs(bb)
            lane = lax.iota(jnp.int32, 16)
            for g in range(C // 16):
                cv = jnp.zeros((16,), jnp.float32)
                for t in range(16):
                    r = g * 16 + t
                    acc = srows[r, pl.ds(0, 16)] * drows[r, pl.ds(0, 16)]
                    for j in range(1, D // 16):
                        acc = acc + (srows[r, pl.ds(j * 16, 16)] *
                                     drows[r, pl.ds(j * 16, 16)])
                    cv = jnp.where(lane == t,
                                   jnp.full((16,), jnp.sum(acc), jnp.float32),
                                   cv)
                ebuf[pl.ds(g * 16, 16)] = jnp.exp(bb * cv + mb)
            pltpu.sync_copy(ebuf, e_out.at[pl.ds(base, C)])
            pltpu.sync_copy(ebuf, denom_sp.at[didx], add=True)
            return carry

        lax.fori_loop(0, NCHUNK, chunk, 0)
        plsc.subcore_barrier()
        pltpu.sync_copy(denom_sp.at[pl.ds(s * zslice, zslice)],
                        denom_out.at[c, pl.ds(s * zslice, zslice)])

    return k(xn, src, dst, bvec)


def _edge_pass2(x, src, dst, e_arr, dinv):
    @functools.partial(
        pl.kernel,
        out_type=jax.ShapeDtypeStruct((2, NPAD, D), jnp.float32),
        mesh=_mesh,
        compiler_params=pltpu.CompilerParams(needs_layout_passes=False),
        scratch_types=[
            pltpu.VMEM((C,), jnp.int32),
            pltpu.VMEM((C,), jnp.int32),
            pltpu.VMEM((C, D), jnp.float32),
            pltpu.VMEM((C,), jnp.float32),
            pltpu.VMEM((NPAD,), jnp.float32),
            pltpu.VMEM((32, D), jnp.float32),
            pltpu.VMEM_SHARED((NPAD, D), jnp.float32),
            pltpu.SemaphoreType.DMA,
        ],
    )
    def k(x_h, src_h, dst_h, e_h, dinv_h, out_h,
          sidx, didx, xrows, ebuf, dinv_v, zbuf, out_sp, sem1):
        c = lax.axis_index("c")
        s = lax.axis_index("s")
        wid = s * NC + c

        for r in range(32):
            for j in range(D // 16):
                zbuf[r, pl.ds(j * 16, 16)] = jnp.zeros((16,), jnp.float32)
        for i in range(ROWS_PER_TILE // 32):
            pltpu.sync_copy(zbuf,
                            out_sp.at[pl.ds(s * ROWS_PER_TILE + i * 32, 32)])
        pltpu.sync_copy(dinv_h, dinv_v)
        plsc.subcore_barrier()

        def chunk(kk, carry):
            base = wid * EPW + kk * C
            pltpu.sync_copy(src_h.at[pl.ds(base, C)], sidx)
            pltpu.sync_copy(dst_h.at[pl.ds(base, C)], didx)
            pltpu.sync_copy(e_h.at[pl.ds(base, C)], ebuf)
            pltpu.async_copy(x_h.at[sidx], xrows, sem1).wait()
            for g in range(C // 16):
                dvec = didx[pl.ds(g * 16, 16)]
                dv = plsc.load_gather(dinv_v, [dvec])
                att16 = ebuf[pl.ds(g * 16, 16)] * dv
                for t in range(16):
                    r = g * 16 + t
                    av = jnp.full((16,), att16[t], jnp.float32)
                    for j in range(D // 16):
                        xrows[r, pl.ds(j * 16, 16)] = (
                            xrows[r, pl.ds(j * 16, 16)] * av)
            pltpu.sync_copy(xrows, out_sp.at[didx], add=True)
            return carry

        lax.fori_loop(0, NCHUNK, chunk, 0)
        plsc.subcore_barrier()
        pltpu.sync_copy(out_sp.at[pl.ds(s * ROWS_PER_TILE, ROWS_PER_TILE)],
                        out_h.at[c, pl.ds(s * ROWS_PER_TILE, ROWS_PER_TILE)])

    return k(x, src, dst, e_arr, dinv)


def kernel(x, edge_index, beta1, beta2):
    src = edge_index[0].astype(jnp.int32)
    dst = edge_index[1].astype(jnp.int32)
    bv1 = jnp.full((16,), beta1, jnp.float32)
    bv2 = jnp.full((16,), beta2, jnp.float32)

    xn1 = _normalize(x)
    e1, d1 = _edge_pass1(xn1, src, dst, bv1)
    o1 = _edge_pass2(x, src, dst, e1, _dinv(d1))
    h1, xn2 = _combine_leaky_normalize(o1[0], o1[1])
    e2, d2 = _edge_pass1(xn2, src, dst, bv2)
    o2 = _edge_pass2(h1, src, dst, e2, _dinv(d2))
    return _combine_leaky(o2[0], o2[1])[:N]


# speedup vs baseline: 10.0444x; 1.9437x over previous
import functools

import jax
import jax.numpy as jnp
import numpy as np
from jax import lax
from jax.experimental import pallas as pl
from jax.experimental.pallas import tpu as pltpu
from jax.experimental.pallas import tpu_sc as plsc

N = 10000
E = 320000
D = 128
NPAD = 10240

NC = 2
NS = 16
NW = NC * NS
EPW = E // NW
C = 80
NCHUNK = EPW // C
ROWS_PER_TILE = NPAD // NS

_mesh = plsc.VectorSubcoreMesh(core_axis_name="c", subcore_axis_name="s")


def _norm_body(x_ref, o_ref):
    xb = x_ref[...]
    n = jnp.sqrt(jnp.sum(xb * xb, axis=1, keepdims=True))
    o_ref[...] = xb / (n + 1e-12)


def _normalize(x):
    return pl.pallas_call(
        _norm_body,
        out_shape=jax.ShapeDtypeStruct((N, D), jnp.float32),
        grid=(25,),
        in_specs=[pl.BlockSpec((400, D), lambda i: (i, 0))],
        out_specs=pl.BlockSpec((400, D), lambda i: (i, 0)),
    )(x)


def _comb_norm_body(a_ref, b_ref, di_ref, h_ref, hn_ref):
    h = (a_ref[...] + b_ref[...]) * di_ref[...]
    h = jnp.where(h > 0, h, 0.3 * h)
    h_ref[...] = h
    n = jnp.sqrt(jnp.sum(h * h, axis=1, keepdims=True))
    hn_ref[...] = h / (n + 1e-12)


def _combine_leaky_normalize(a, b, dinv):
    return pl.pallas_call(
        _comb_norm_body,
        out_shape=(jax.ShapeDtypeStruct((NPAD, D), jnp.float32),
                   jax.ShapeDtypeStruct((NPAD, D), jnp.float32)),
        grid=(20,),
        in_specs=[pl.BlockSpec((512, D), lambda i: (i, 0)),
                  pl.BlockSpec((512, D), lambda i: (i, 0)),
                  pl.BlockSpec((512, 1), lambda i: (i, 0))],
        out_specs=(pl.BlockSpec((512, D), lambda i: (i, 0)),
                   pl.BlockSpec((512, D), lambda i: (i, 0))),
    )(a, b, dinv)


def _comb_body(a_ref, b_ref, di_ref, o_ref):
    h = (a_ref[...] + b_ref[...]) * di_ref[...]
    o_ref[...] = jnp.where(h > 0, h, 0.3 * h)


def _combine_leaky(a, b, dinv):
    return pl.pallas_call(
        _comb_body,
        out_shape=jax.ShapeDtypeStruct((NPAD, D), jnp.float32),
        grid=(20,),
        in_specs=[pl.BlockSpec((512, D), lambda i: (i, 0)),
                  pl.BlockSpec((512, D), lambda i: (i, 0)),
                  pl.BlockSpec((512, 1), lambda i: (i, 0))],
        out_specs=pl.BlockSpec((512, D), lambda i: (i, 0)),
    )(a, b, dinv)


def _dinv_body(d_ref, o_ref):
    o_ref[...] = 1.0 / (d_ref[0] + d_ref[1] + 1e-12)


def _dinv(denom2):
    d3 = denom2.reshape(2, 80, 128)
    out = pl.pallas_call(
        _dinv_body,
        out_shape=jax.ShapeDtypeStruct((80, 128), jnp.float32),
    )(d3)
    return out.reshape(NPAD, 1)


def _edge_pass1(xn, src3, dst3, bvec):
    @functools.partial(
        pl.kernel,
        out_type=(jax.ShapeDtypeStruct((NW, NCHUNK, C), jnp.float32),
                  jax.ShapeDtypeStruct((2, NPAD), jnp.float32)),
        mesh=_mesh,
        compiler_params=pltpu.CompilerParams(needs_layout_passes=False),
        scratch_types=[
            [pltpu.VMEM((1, C), jnp.int32)] * 4,
            [pltpu.VMEM((1, C), jnp.int32)] * 4,
            pltpu.VMEM((NCHUNK, C), jnp.float32),
            pltpu.VMEM((C, D), jnp.float32),
            pltpu.VMEM((C, D), jnp.float32),
            pltpu.VMEM((C, D), jnp.float32),
            pltpu.VMEM((C, D), jnp.float32),
            pltpu.VMEM((16,), jnp.float32),
            pltpu.VMEM((NPAD // NS,), jnp.float32),
            pltpu.VMEM_SHARED((NPAD,), jnp.float32),
            [pltpu.SemaphoreType.DMA] * 4,
            pltpu.SemaphoreType.DMA,
            pltpu.SemaphoreType.DMA,
            pltpu.SemaphoreType.DMA,
            pltpu.SemaphoreType.DMA,
            [pltpu.SemaphoreType.DMA] * 4,
        ],
    )
    def k(xn_h, src_h, dst_h, bvec_h, e_out, denom_out,
          sidx, didx, ebuf_all, srows0, srows1, drows0, drows1,
          bv, zbuf, denom_sp, sem_i, sem_s0, sem_s1, sem_d0, sem_d1, sem_e):
        cc = lax.axis_index("c")
        s = lax.axis_index("s")
        wid = s * NC + cc

        srows = (srows0, srows1)
        drows = (drows0, drows1)
        sem_s = (sem_s0, sem_s1)
        sem_d = (sem_d0, sem_d1)

        zslice = NPAD // NS
        for i in range(zslice // 16):
            zbuf[pl.ds(i * 16, 16)] = jnp.zeros((16,), jnp.float32)
        pltpu.sync_copy(zbuf, denom_sp.at[pl.ds(s * zslice, zslice)])
        pltpu.sync_copy(bvec_h, bv)
        plsc.subcore_barrier()

        def issue_idx(a, q):
            pltpu.async_copy(src_h.at[wid, pl.ds(a, 1)], sidx[q], sem_i[q])
            pltpu.async_copy(dst_h.at[wid, pl.ds(a, 1)], didx[q], sem_i[q])

        def wait_idx(a, q):
            pltpu.make_async_copy(src_h.at[wid, pl.ds(a, 1)], sidx[q],
                                  sem_i[q]).wait()
            pltpu.make_async_copy(dst_h.at[wid, pl.ds(a, 1)], didx[q],
                                  sem_i[q]).wait()

        def issue_g(p, q):
            pltpu.async_copy(xn_h.at[sidx[q].at[0]], srows[p], sem_s[p])
            pltpu.async_copy(xn_h.at[didx[q].at[0]], drows[p], sem_d[p])

        def wait_g(p, q):
            pltpu.make_async_copy(xn_h.at[sidx[q].at[0]], srows[p],
                                  sem_s[p]).wait()
            pltpu.make_async_copy(xn_h.at[didx[q].at[0]], drows[p],
                                  sem_d[p]).wait()

        def issue_e(a, q):
            pltpu.async_copy(ebuf_all.at[a], denom_sp.at[didx[q].at[0]],
                             sem_e[q], add=True)

        def wait_e(a, q):
            pltpu.make_async_copy(ebuf_all.at[a], denom_sp.at[didx[q].at[0]],
                                  sem_e[q]).wait()

        def compute(a, p):
            bb = bv[...]
            mb = -jnp.abs(bb)
            lane = lax.iota(jnp.int32, 16)
            sp, dp = srows[p], drows[p]

            def grp(g, carry):
                cv = jnp.zeros((16,), jnp.float32)
                for t in range(16):
                    r = g * 16 + t
                    acc = sp[r, pl.ds(0, 16)] * dp[r, pl.ds(0, 16)]
                    for j in range(1, D // 16):
                        acc = acc + (sp[r, pl.ds(j * 16, 16)] *
                                     dp[r, pl.ds(j * 16, 16)])
                    cv = jnp.where(lane == t,
                                   jnp.full((16,), jnp.sum(acc), jnp.float32),
                                   cv)
                ebuf_all[a, pl.ds(g * 16, 16)] = jnp.exp(bb * cv + mb)
                return carry

            lax.fori_loop(0, C // 16, grp, 0)

        def half(a, p, q):
            @pl.when((a >= 0) & (a < NCHUNK))
            def _():
                wait_g(p, q)

                @pl.when(a >= 2)
                def _():
                    wait_e(a - 2, (q + 2) % 4)

                @pl.when(a + 2 < NCHUNK)
                def _():
                    issue_idx(a + 2, (q + 2) % 4)

                compute(a, p)
                issue_e(a, q)

                @pl.when(a + 2 < NCHUNK)
                def _():
                    wait_idx(a + 2, (q + 2) % 4)
                    issue_g(p, (q + 2) % 4)

        issue_idx(0, 0)
        issue_idx(1, 1)
        wait_idx(0, 0)
        wait_idx(1, 1)
        issue_g(0, 0)
        issue_g(1, 1)

        def quad(i, carry):
            half(4 * i - 2, 0, 2)
            half(4 * i - 1, 1, 3)
            half(4 * i, 0, 0)
            half(4 * i + 1, 1, 1)
            return carry

        lax.fori_loop(0, (NCHUNK + 5) // 4, quad, 0)
        wait_e(NCHUNK - 2, 3)
        wait_e(NCHUNK - 1, 0)

        plsc.subcore_barrier()
        pltpu.sync_copy(denom_sp.at[pl.ds(s * zslice, zslice)],
                        denom_out.at[cc, pl.ds(s * zslice, zslice)])
        pltpu.sync_copy(ebuf_all, e_out.at[wid])

    return k(xn, src3, dst3, bvec)


def _edge_pass2(x, src3, dst3, e3):
    @functools.partial(
        pl.kernel,
        out_type=jax.ShapeDtypeStruct((2, NPAD, D), jnp.float32),
        mesh=_mesh,
        compiler_params=pltpu.CompilerParams(needs_layout_passes=False),
        scratch_types=[
            [pltpu.VMEM((1, C), jnp.int32)] * 4,
            [pltpu.VMEM((1, C), jnp.int32)] * 4,
            [pltpu.VMEM((1, C), jnp.float32)] * 4,
            pltpu.VMEM((C, D), jnp.float32),
            pltpu.VMEM((C, D), jnp.float32),
            pltpu.VMEM((C, D), jnp.float32),
            pltpu.VMEM((C, D), jnp.float32),
            pltpu.VMEM((8, D), jnp.float32),
            pltpu.VMEM_SHARED((NPAD, D), jnp.float32),
            [pltpu.SemaphoreType.DMA] * 4,
            pltpu.SemaphoreType.DMA,
            pltpu.SemaphoreType.DMA,
            pltpu.SemaphoreType.DMA,
            pltpu.SemaphoreType.DMA,
        ],
    )
    def k(x_h, src_h, dst_h, e_h, out_h,
          sidx, didx, ebuf, xrows0, xrows1, mrows0, mrows1,
          zbuf, out_sp, sem_i, sem_x0, sem_x1, sem_m0, sem_m1):
        cc = lax.axis_index("c")
        s = lax.axis_index("s")
        wid = s * NC + cc

        xrows = (xrows0, xrows1)
        mrows = (mrows0, mrows1)
        sem_x = (sem_x0, sem_x1)
        sem_m = (sem_m0, sem_m1)

        for r in range(8):
            for j in range(D // 16):
                zbuf[r, pl.ds(j * 16, 16)] = jnp.zeros((16,), jnp.float32)
        for i in range(ROWS_PER_TILE // 8):
            pltpu.sync_copy(zbuf,
                            out_sp.at[pl.ds(s * ROWS_PER_TILE + i * 8, 8)])
        plsc.subcore_barrier()

        def issue_idx(a, q):
            pltpu.async_copy(src_h.at[wid, pl.ds(a, 1)], sidx[q], sem_i[q])
            pltpu.async_copy(dst_h.at[wid, pl.ds(a, 1)], didx[q], sem_i[q])
            pltpu.async_copy(e_h.at[wid, pl.ds(a, 1)], ebuf[q], sem_i[q])

        def wait_idx(a, q):
            pltpu.make_async_copy(src_h.at[wid, pl.ds(a, 1)], sidx[q],
                                  sem_i[q]).wait()
            pltpu.make_async_copy(dst_h.at[wid, pl.ds(a, 1)], didx[q],
                                  sem_i[q]).wait()
            pltpu.make_async_copy(e_h.at[wid, pl.ds(a, 1)], ebuf[q],
                                  sem_i[q]).wait()

        def issue_g(p, q):
            pltpu.async_copy(x_h.at[sidx[q].at[0]], xrows[p], sem_x[p])

        def wait_g(p, q):
            pltpu.make_async_copy(x_h.at[sidx[q].at[0]], xrows[p],
                                  sem_x[p]).wait()

        def issue_sc(p, q):
            pltpu.async_copy(mrows[p], out_sp.at[didx[q].at[0]], sem_m[p],
                             add=True)

        def wait_sc(p, q):
            pltpu.make_async_copy(mrows[p], out_sp.at[didx[q].at[0]],
                                  sem_m[p]).wait()

        def compute(p, q):
            xp, mp = xrows[p], mrows[p]
            eb = ebuf[q]

            def grp(g, carry):
                att16 = eb[0, pl.ds(g * 16, 16)]
                for t in range(16):
                    r = g * 16 + t
                    av = jnp.full((16,), att16[t], jnp.float32)
                    for j in range(D // 16):
                        mp[r, pl.ds(j * 16, 16)] = (
                            xp[r, pl.ds(j * 16, 16)] * av)
                return carry

            lax.fori_loop(0, C // 16, grp, 0)

        def half(a, p, q):
            @pl.when((a >= 0) & (a < NCHUNK))
            def _():
                wait_g(p, q)

                @pl.when(a >= 2)
                def _():
                    wait_sc(p, (q + 2) % 4)

                @pl.when(a + 2 < NCHUNK)
                def _():
                    issue_idx(a + 2, (q + 2) % 4)

                compute(p, q)
                issue_sc(p, q)

                @pl.when(a + 2 < NCHUNK)
                def _():
                    wait_idx(a + 2, (q + 2) % 4)
                    issue_g(p, (q + 2) % 4)

        issue_idx(0, 0)
        issue_idx(1, 1)
        wait_idx(0, 0)
        wait_idx(1, 1)
        issue_g(0, 0)
        issue_g(1, 1)

        def quad(i, carry):
            half(4 * i - 2, 0, 2)
            half(4 * i - 1, 1, 3)
            half(4 * i, 0, 0)
            half(4 * i + 1, 1, 1)
            return carry

        lax.fori_loop(0, (NCHUNK + 5) // 4, quad, 0)
        wait_sc(1, 3)
        wait_sc(0, 0)

        plsc.subcore_barrier()

        def writeback(i, carry):
            off = s * ROWS_PER_TILE + i * 80
            pltpu.sync_copy(out_sp.at[pl.ds(off, 80)],
                            out_h.at[cc, pl.ds(off, 80)])
            return carry

        lax.fori_loop(0, ROWS_PER_TILE // 80, writeback, 0)

    return k(x, src3, dst3, e3)


def kernel(x, edge_index, beta1, beta2):
    src3 = edge_index[0].astype(jnp.int32).reshape(NW, NCHUNK, C)
    dst3 = edge_index[1].astype(jnp.int32).reshape(NW, NCHUNK, C)
    bv1 = jnp.full((16,), beta1, jnp.float32)
    bv2 = jnp.full((16,), beta2, jnp.float32)

    xn1 = _normalize(x)
    e1, d1 = _edge_pass1(xn1, src3, dst3, bv1)
    o1 = _edge_pass2(x, src3, dst3, e1)
    h1, xn2 = _combine_leaky_normalize(o1[0], o1[1], _dinv(d1))
    e2, d2 = _edge_pass1(xn2, src3, dst3, bv2)
    o2 = _edge_pass2(h1, src3, dst3, e2)
    return _combine_leaky(o2[0], o2[1], _dinv(d2))[:N]
